# Initial kernel scaffold; baseline (speedup 1.0000x reference)
import jax
import jax.numpy as jnp
from jax.experimental import pallas as pl


def kernel(boxes, scores):
    raise NotImplementedError("write your pallas kernel here")


# speedup vs baseline: 5.3137x; 5.3137x over previous
import jax
import jax.numpy as jnp
from jax import lax
from jax.experimental import pallas as pl
from jax.experimental.pallas import tpu as pltpu, tpu_sc as plsc

N = 20000
MAX_DET = 100
IOU_THRESH = 0.5
NSUB = 16
L = 16
NPAD = 20480
PER = NPAD // NSUB
NVEC = PER // L

_SUPPRESSED = -1e9
_PAD_SCORE = -3e9
_PAD_COORD = 2e9
_NEG_HUGE = -3e38


def _nms_kernel(x1h, y1h, x2h, y2h, sh,
                out_h,
                x1v, y1v, x2v, y2v, areab, sm,
                cand, tb, rec, outr):
    cid = lax.axis_index("c")
    sid = lax.axis_index("s")
    base = sid * PER

    rec[:] = jnp.zeros((L,), jnp.float32)
    pltpu.sync_copy(rec, cand.at[cid, 0, sid])
    pltpu.sync_copy(rec, cand.at[cid, 1, sid])

    pltpu.sync_copy(x1h.at[pl.ds(base, PER)], x1v)
    pltpu.sync_copy(y1h.at[pl.ds(base, PER)], y1v)
    pltpu.sync_copy(x2h.at[pl.ds(base, PER)], x2v)
    pltpu.sync_copy(y2h.at[pl.ds(base, PER)], y2v)
    pltpu.sync_copy(sh.at[pl.ds(base, PER)], sm)

    lanes = lax.iota(jnp.int32, L)

    def init_area(j, c):
        s = pl.ds(j * L, L)
        areab[s] = (x2v[s] - x1v[s]) * (y2v[s] - y1v[s])
        return c

    lax.fori_loop(0, NVEC, init_area, 0)

    col = [jnp.full((L,), c, jnp.int32) for c in range(16)]

    def step(k, carry):
        sx1, sy1, sx2, sy2, sarea, sidx = carry
        ph = lax.rem(k, 2)

        def pass_body(j, rc):
            rmax, ridx = rc
            s = pl.ds(j * L, L)
            bx1 = x1v[s]
            by1 = y1v[s]
            bx2 = x2v[s]
            by2 = y2v[s]
            ab = areab[s]
            sv = sm[s]
            ix1 = jnp.maximum(sx1, bx1)
            iy1 = jnp.maximum(sy1, by1)
            ix2 = jnp.minimum(sx2, bx2)
            iy2 = jnp.minimum(sy2, by2)
            inter = jnp.maximum(ix2 - ix1, 0.0) * jnp.maximum(iy2 - iy1, 0.0)
            iou = inter / (sarea + ab - inter + 1e-8)
            idxv = lanes + (base + j * L)
            supp = (iou > IOU_THRESH) | (idxv == sidx)
            ns = jnp.where(supp, _SUPPRESSED, sv)
            sm[s] = ns
            better = ns > rmax
            rmax = jnp.where(better, ns, rmax)
            ridx = jnp.where(better, idxv, ridx)
            return (rmax, ridx)

        rmax0 = jnp.full((L,), _NEG_HUGE, jnp.float32)
        ridx0 = jnp.zeros((L,), jnp.int32)
        rmax, ridx = lax.fori_loop(0, NVEC, pass_body, (rmax0, ridx0))

        mloc = jnp.max(rmax)
        iloc = jnp.min(jnp.where(rmax == mloc, ridx, jnp.int32(2**31 - 1)))
        li_vec = jnp.zeros((L,), jnp.int32) + (iloc - base)
        vkf = (k + 1).astype(jnp.float32)

        rv = jnp.where(lanes == 0, mloc, 0.0)
        rv = jnp.where(lanes == 1, iloc.astype(jnp.float32), rv)
        rv = jnp.where(lanes == 2, plsc.load_gather(x1v, [li_vec]), rv)
        rv = jnp.where(lanes == 3, plsc.load_gather(y1v, [li_vec]), rv)
        rv = jnp.where(lanes == 4, plsc.load_gather(x2v, [li_vec]), rv)
        rv = jnp.where(lanes == 5, plsc.load_gather(y2v, [li_vec]), rv)
        rv = jnp.where(lanes == 6, plsc.load_gather(areab, [li_vec]), rv)
        rv = jnp.where((lanes == 7) | (lanes == 8) | (lanes == 15), vkf, rv)
        rec[:] = rv
        pltpu.sync_copy(rec, cand.at[cid, ph, sid])

        def spin_body(st):
            pltpu.sync_copy(cand.at[cid, ph], tb)
            v7 = plsc.load_gather(tb, [lanes, col[7]])
            v8 = plsc.load_gather(tb, [lanes, col[8]])
            v15 = plsc.load_gather(tb, [lanes, col[15]])
            okv = (v7 == vkf) & (v8 == vkf) & (v15 == vkf)
            return jnp.min(jnp.where(okv, jnp.int32(1), jnp.int32(0)))

        lax.while_loop(lambda st: st == 0, spin_body, jnp.int32(0))

        valc = plsc.load_gather(tb, [lanes, col[0]])
        idxc = plsc.load_gather(tb, [lanes, col[1]])
        m = jnp.max(valc)
        win = valc == m
        gidxf = jnp.min(jnp.where(win, idxc, 3e38))
        gidx = gidxf.astype(jnp.int32)
        sel = idxc == gidxf
        wx1 = jnp.max(jnp.where(sel, plsc.load_gather(tb, [lanes, col[2]]),
                                _NEG_HUGE))
        wy1 = jnp.max(jnp.where(sel, plsc.load_gather(tb, [lanes, col[3]]),
                                _NEG_HUGE))
        wx2 = jnp.max(jnp.where(sel, plsc.load_gather(tb, [lanes, col[4]]),
                                _NEG_HUGE))
        wy2 = jnp.max(jnp.where(sel, plsc.load_gather(tb, [lanes, col[5]]),
                                _NEG_HUGE))
        wa = jnp.max(jnp.where(sel, plsc.load_gather(tb, [lanes, col[6]]),
                               _NEG_HUGE))

        @pl.when((cid == 0) & (sid == 0))
        def _():
            bv = jnp.where(lanes == 0, wx1, 0.0)
            bv = jnp.where(lanes == 1, wy1, bv)
            bv = jnp.where(lanes == 2, wx2, bv)
            bv = jnp.where(lanes == 3, wy2, bv)
            bv = jnp.where(lanes == 4, jnp.where(m > -1e8, m, 0.0), bv)
            bv = jnp.where(lanes == 5, gidxf, bv)
            outr[pl.ds(k * L, L)] = bv

        return (wx1, wy1, wx2, wy2, wa, gidx)

    init = (jnp.float32(3e9), jnp.float32(3e9), jnp.float32(3e9),
            jnp.float32(3e9), jnp.float32(0.0), jnp.int32(-1))
    lax.fori_loop(0, MAX_DET, step, init)

    @pl.when((cid == 0) & (sid == 0))
    def _():
        pltpu.sync_copy(outr, out_h)


@jax.jit
def kernel(boxes, scores):
    padc = jnp.full((NPAD - N,), _PAD_COORD, jnp.float32)
    x1 = jnp.concatenate([boxes[:, 0], padc])
    y1 = jnp.concatenate([boxes[:, 1], padc])
    x2 = jnp.concatenate([boxes[:, 2], padc])
    y2 = jnp.concatenate([boxes[:, 3], padc])
    sp = jnp.concatenate([scores,
                          jnp.full((NPAD - N,), _PAD_SCORE, jnp.float32)])

    mesh = plsc.VectorSubcoreMesh(core_axis_name="c", subcore_axis_name="s",
                                  num_cores=2, num_subcores=NSUB)
    run = pl.kernel(
        _nms_kernel,
        out_type=jax.ShapeDtypeStruct((MAX_DET * L,), jnp.float32),
        mesh=mesh,
        compiler_params=pltpu.CompilerParams(needs_layout_passes=False),
        scratch_types=[
            pltpu.VMEM((PER,), jnp.float32),
            pltpu.VMEM((PER,), jnp.float32),
            pltpu.VMEM((PER,), jnp.float32),
            pltpu.VMEM((PER,), jnp.float32),
            pltpu.VMEM((PER,), jnp.float32),
            pltpu.VMEM((PER,), jnp.float32),
            pltpu.HBM((2, 2, NSUB, L), jnp.float32),
            pltpu.VMEM((NSUB, L), jnp.float32),
            pltpu.VMEM((L,), jnp.float32),
            pltpu.VMEM((MAX_DET * L,), jnp.float32),
        ],
    )
    r = run(x1, y1, x2, y2, sp).reshape(MAX_DET, L)
    return r[:, :4], r[:, 4], r[:, 5].astype(jnp.int32)


# speedup vs baseline: 5.4863x; 1.0325x over previous
import jax
import jax.numpy as jnp
from jax import lax
from jax.experimental import pallas as pl
from jax.experimental.pallas import tpu as pltpu, tpu_sc as plsc

N = 20000
MAX_DET = 100
IOU_THRESH = 0.5
NSUB = 16
L = 16
NPAD = 20480
PER = NPAD // NSUB
NVEC = PER // L

_SUPPRESSED = -1e9
_PAD_SCORE = -3e9
_PAD_COORD = 2e9
_NEG_HUGE = -3e38


def _nms_kernel(x1h, y1h, x2h, y2h, sh,
                out_h,
                x1v, y1v, x2v, y2v, areab, sm,
                cand, tb, rec, outr, psem):
    cid = lax.axis_index("c")
    sid = lax.axis_index("s")
    base = sid * PER

    rec[:] = jnp.zeros((L,), jnp.float32)
    pltpu.sync_copy(rec, cand.at[cid, 0, sid])
    pltpu.sync_copy(rec, cand.at[cid, 1, sid])

    pltpu.sync_copy(x1h.at[pl.ds(base, PER)], x1v)
    pltpu.sync_copy(y1h.at[pl.ds(base, PER)], y1v)
    pltpu.sync_copy(x2h.at[pl.ds(base, PER)], x2v)
    pltpu.sync_copy(y2h.at[pl.ds(base, PER)], y2v)
    pltpu.sync_copy(sh.at[pl.ds(base, PER)], sm)

    lanes = lax.iota(jnp.int32, L)

    def init_area(j, c):
        s = pl.ds(j * L, L)
        areab[s] = (x2v[s] - x1v[s]) * (y2v[s] - y1v[s])
        return c

    lax.fori_loop(0, NVEC, init_area, 0)

    col = [jnp.full((L,), c, jnp.int32) for c in range(16)]

    def step(k, carry):
        sx1, sy1, sx2, sy2, sarea, sidx = carry
        ph = lax.rem(k, 2)

        def pass_body(jo, rc):
            rmax, ridx = rc
            for ji in range(4):
                j = jo * 4 + ji
                s = pl.ds(j * L, L)
                bx1 = x1v[s]
                by1 = y1v[s]
                bx2 = x2v[s]
                by2 = y2v[s]
                ab = areab[s]
                sv = sm[s]
                ix1 = jnp.maximum(sx1, bx1)
                iy1 = jnp.maximum(sy1, by1)
                ix2 = jnp.minimum(sx2, bx2)
                iy2 = jnp.minimum(sy2, by2)
                inter = (jnp.maximum(ix2 - ix1, 0.0)
                         * jnp.maximum(iy2 - iy1, 0.0))
                iou = inter / (sarea + ab - inter + 1e-8)
                ns = jnp.where(iou > IOU_THRESH, _SUPPRESSED, sv)
                sm[s] = ns
                better = ns > rmax
                rmax = jnp.where(better, ns, rmax)
                ridx = jnp.where(better, lanes + (base + j * L), ridx)
            return (rmax, ridx)

        rmax0 = jnp.full((L,), _NEG_HUGE, jnp.float32)
        ridx0 = jnp.zeros((L,), jnp.int32)
        rmax, ridx = lax.fori_loop(0, NVEC // 4, pass_body, (rmax0, ridx0))

        mloc = jnp.max(rmax)
        iloc = jnp.min(jnp.where(rmax == mloc, ridx, jnp.int32(2**31 - 1)))
        li_vec = jnp.zeros((L,), jnp.int32) + (iloc - base)
        vkf = (k + 1).astype(jnp.float32)

        rv = jnp.where(lanes == 0, mloc, 0.0)
        rv = jnp.where(lanes == 1, iloc.astype(jnp.float32), rv)
        rv = jnp.where(lanes == 2, plsc.load_gather(x1v, [li_vec]), rv)
        rv = jnp.where(lanes == 3, plsc.load_gather(y1v, [li_vec]), rv)
        rv = jnp.where(lanes == 4, plsc.load_gather(x2v, [li_vec]), rv)
        rv = jnp.where(lanes == 5, plsc.load_gather(y2v, [li_vec]), rv)
        rv = jnp.where(lanes == 6, plsc.load_gather(areab, [li_vec]), rv)
        rv = jnp.where((lanes == 7) | (lanes == 8) | (lanes == 15), vkf, rv)
        rec[:] = rv
        pub = pltpu.make_async_copy(rec, cand.at[cid, ph, sid], psem)
        pub.start()

        def spin_body(st):
            pltpu.sync_copy(cand.at[cid, ph], tb)
            v7 = plsc.load_gather(tb, [lanes, col[7]])
            v8 = plsc.load_gather(tb, [lanes, col[8]])
            v15 = plsc.load_gather(tb, [lanes, col[15]])
            okv = (v7 == vkf) & (v8 == vkf) & (v15 == vkf)
            return jnp.min(jnp.where(okv, jnp.int32(1), jnp.int32(0)))

        lax.while_loop(lambda st: st == 0, spin_body, jnp.int32(0))
        pub.wait()

        valc = plsc.load_gather(tb, [lanes, col[0]])
        idxc = plsc.load_gather(tb, [lanes, col[1]])
        m = jnp.max(valc)
        win = valc == m
        gidxf = jnp.min(jnp.where(win, idxc, 3e38))
        gidx = gidxf.astype(jnp.int32)
        sel = idxc == gidxf
        wx1 = jnp.max(jnp.where(sel, plsc.load_gather(tb, [lanes, col[2]]),
                                _NEG_HUGE))
        wy1 = jnp.max(jnp.where(sel, plsc.load_gather(tb, [lanes, col[3]]),
                                _NEG_HUGE))
        wx2 = jnp.max(jnp.where(sel, plsc.load_gather(tb, [lanes, col[4]]),
                                _NEG_HUGE))
        wy2 = jnp.max(jnp.where(sel, plsc.load_gather(tb, [lanes, col[5]]),
                                _NEG_HUGE))
        wa = jnp.max(jnp.where(sel, plsc.load_gather(tb, [lanes, col[6]]),
                               _NEG_HUGE))

        @pl.when((cid == 0) & (sid == 0))
        def _():
            bv = jnp.where(lanes == 0, wx1, 0.0)
            bv = jnp.where(lanes == 1, wy1, bv)
            bv = jnp.where(lanes == 2, wx2, bv)
            bv = jnp.where(lanes == 3, wy2, bv)
            bv = jnp.where(lanes == 4, jnp.where(m > -1e8, m, 0.0), bv)
            bv = jnp.where(lanes == 5, gidxf, bv)
            outr[pl.ds(k * L, L)] = bv

        return (wx1, wy1, wx2, wy2, wa, gidx)

    init = (jnp.float32(3e9), jnp.float32(3e9), jnp.float32(3e9),
            jnp.float32(3e9), jnp.float32(0.0), jnp.int32(-1))
    lax.fori_loop(0, MAX_DET, step, init)

    @pl.when((cid == 0) & (sid == 0))
    def _():
        pltpu.sync_copy(outr, out_h)


@jax.jit
def kernel(boxes, scores):
    padc = jnp.full((NPAD - N,), _PAD_COORD, jnp.float32)
    x1 = jnp.concatenate([boxes[:, 0], padc])
    y1 = jnp.concatenate([boxes[:, 1], padc])
    x2 = jnp.concatenate([boxes[:, 2], padc])
    y2 = jnp.concatenate([boxes[:, 3], padc])
    sp = jnp.concatenate([scores,
                          jnp.full((NPAD - N,), _PAD_SCORE, jnp.float32)])

    mesh = plsc.VectorSubcoreMesh(core_axis_name="c", subcore_axis_name="s",
                                  num_cores=2, num_subcores=NSUB)
    run = pl.kernel(
        _nms_kernel,
        out_type=jax.ShapeDtypeStruct((MAX_DET * L,), jnp.float32),
        mesh=mesh,
        compiler_params=pltpu.CompilerParams(needs_layout_passes=False),
        scratch_types=[
            pltpu.VMEM((PER,), jnp.float32),
            pltpu.VMEM((PER,), jnp.float32),
            pltpu.VMEM((PER,), jnp.float32),
            pltpu.VMEM((PER,), jnp.float32),
            pltpu.VMEM((PER,), jnp.float32),
            pltpu.VMEM((PER,), jnp.float32),
            pltpu.HBM((2, 2, NSUB, L), jnp.float32),
            pltpu.VMEM((NSUB, L), jnp.float32),
            pltpu.VMEM((L,), jnp.float32),
            pltpu.VMEM((MAX_DET * L,), jnp.float32),
            pltpu.SemaphoreType.DMA,
        ],
    )
    r = run(x1, y1, x2, y2, sp).reshape(MAX_DET, L)
    return r[:, :4], r[:, 4], r[:, 5].astype(jnp.int32)


# speedup vs baseline: 5.5440x; 1.0105x over previous
import jax
import jax.numpy as jnp
from jax import lax
from jax.experimental import pallas as pl
from jax.experimental.pallas import tpu as pltpu, tpu_sc as plsc

N = 20000
MAX_DET = 100
IOU_THRESH = 0.5
NSUB = 16
L = 16
NPAD = 20480
PER = NPAD // NSUB
NVEC = PER // L

_SUPPRESSED = -1e9
_PAD_SCORE = -3e9
_PAD_COORD = 2e9
_NEG_HUGE = -3e38


def _nms_kernel(x1h, y1h, x2h, y2h, sh,
                out_h,
                x1v, y1v, x2v, y2v, areab, sm,
                cand, tb, rec, outr, psem):
    cid = lax.axis_index("c")
    sid = lax.axis_index("s")
    base = sid * PER

    rec[:] = jnp.zeros((L,), jnp.float32)
    pltpu.sync_copy(rec, cand.at[cid, 0, sid])
    pltpu.sync_copy(rec, cand.at[cid, 1, sid])

    pltpu.sync_copy(x1h.at[pl.ds(base, PER)], x1v)
    pltpu.sync_copy(y1h.at[pl.ds(base, PER)], y1v)
    pltpu.sync_copy(x2h.at[pl.ds(base, PER)], x2v)
    pltpu.sync_copy(y2h.at[pl.ds(base, PER)], y2v)
    pltpu.sync_copy(sh.at[pl.ds(base, PER)], sm)

    lanes = lax.iota(jnp.int32, L)

    def init_area(j, c):
        s = pl.ds(j * L, L)
        areab[s] = (x2v[s] - x1v[s]) * (y2v[s] - y1v[s])
        return c

    lax.fori_loop(0, NVEC, init_area, 0)

    col = [jnp.full((L,), c, jnp.int32) for c in range(16)]

    def step(k, carry):
        sx1, sy1, sx2, sy2, sarea = carry
        ph = lax.rem(k, 2)

        def pass_body(jo, rc):
            rmax, ridx = rc
            for ji in range(4):
                j = jo * 4 + ji
                s = pl.ds(j * L, L)
                bx1 = x1v[s]
                by1 = y1v[s]
                bx2 = x2v[s]
                by2 = y2v[s]
                ab = areab[s]
                sv = sm[s]
                ix1 = jnp.maximum(sx1, bx1)
                iy1 = jnp.maximum(sy1, by1)
                ix2 = jnp.minimum(sx2, bx2)
                iy2 = jnp.minimum(sy2, by2)
                inter = (jnp.maximum(ix2 - ix1, 0.0)
                         * jnp.maximum(iy2 - iy1, 0.0))
                iou = inter / (sarea + ab - inter + 1e-8)
                ns = jnp.where(iou > IOU_THRESH, _SUPPRESSED, sv)
                sm[s] = ns
                better = ns > rmax
                rmax = jnp.where(better, ns, rmax)
                ridx = jnp.where(better, lanes + (base + j * L), ridx)
            return (rmax, ridx)

        rmax0 = jnp.full((L,), _NEG_HUGE, jnp.float32)
        ridx0 = jnp.zeros((L,), jnp.int32)
        rmax, ridx = lax.fori_loop(0, NVEC // 4, pass_body, (rmax0, ridx0))

        mloc = jnp.max(rmax)
        iloc = jnp.min(jnp.where(rmax == mloc, ridx, jnp.int32(2**31 - 1)))
        li_vec = jnp.zeros((L,), jnp.int32) + (iloc - base)
        vkf = (k + 1).astype(jnp.float32)

        rv = jnp.where(lanes == 0, mloc, 0.0)
        rv = jnp.where(lanes == 1, iloc.astype(jnp.float32), rv)
        rv = jnp.where(lanes == 2, plsc.load_gather(x1v, [li_vec]), rv)
        rv = jnp.where(lanes == 3, plsc.load_gather(y1v, [li_vec]), rv)
        rv = jnp.where(lanes == 4, plsc.load_gather(x2v, [li_vec]), rv)
        rv = jnp.where(lanes == 5, plsc.load_gather(y2v, [li_vec]), rv)
        rv = jnp.where(lanes == 6, plsc.load_gather(areab, [li_vec]), rv)
        rv = jnp.where((lanes == 7) | (lanes == 8) | (lanes == 15), vkf, rv)
        rec[:] = rv
        pub = pltpu.make_async_copy(rec, cand.at[cid, ph, sid], psem)
        pub.start()

        def spin_body(st):
            pltpu.sync_copy(cand.at[cid, ph], tb)
            v7 = plsc.load_gather(tb, [lanes, col[7]])
            v8 = plsc.load_gather(tb, [lanes, col[8]])
            v15 = plsc.load_gather(tb, [lanes, col[15]])
            okv = (v7 == vkf) & (v8 == vkf) & (v15 == vkf)
            return jnp.min(jnp.where(okv, jnp.int32(1), jnp.int32(0)))

        lax.while_loop(lambda st: st == 0, spin_body, jnp.int32(0))
        pub.wait()

        valc = plsc.load_gather(tb, [lanes, col[0]])
        idxc = plsc.load_gather(tb, [lanes, col[1]])
        m = jnp.max(valc)
        win = valc == m
        gidxf = jnp.min(jnp.where(win, idxc, 3e38))
        gidx = gidxf.astype(jnp.int32)
        wrow = jnp.zeros((L,), jnp.int32) + gidx // PER
        wx1 = plsc.load_gather(tb, [wrow, col[2]])
        wy1 = plsc.load_gather(tb, [wrow, col[3]])
        wx2 = plsc.load_gather(tb, [wrow, col[4]])
        wy2 = plsc.load_gather(tb, [wrow, col[5]])
        wa = plsc.load_gather(tb, [wrow, col[6]])

        @pl.when((cid == 0) & (sid == 0))
        def _():
            bv = jnp.where(lanes == 0, wx1, 0.0)
            bv = jnp.where(lanes == 1, wy1, bv)
            bv = jnp.where(lanes == 2, wx2, bv)
            bv = jnp.where(lanes == 3, wy2, bv)
            bv = jnp.where(lanes == 4, jnp.where(m > -1e8, m, 0.0), bv)
            bv = jnp.where(lanes == 5, gidxf, bv)
            outr[pl.ds(k * L, L)] = bv

        return (wx1, wy1, wx2, wy2, wa)

    z = jnp.full((L,), 3e9, jnp.float32)
    init = (z, z, z, z, jnp.zeros((L,), jnp.float32))
    lax.fori_loop(0, MAX_DET, step, init)

    @pl.when((cid == 0) & (sid == 0))
    def _():
        pltpu.sync_copy(outr, out_h)


@jax.jit
def kernel(boxes, scores):
    padc = jnp.full((NPAD - N,), _PAD_COORD, jnp.float32)
    x1 = jnp.concatenate([boxes[:, 0], padc])
    y1 = jnp.concatenate([boxes[:, 1], padc])
    x2 = jnp.concatenate([boxes[:, 2], padc])
    y2 = jnp.concatenate([boxes[:, 3], padc])
    sp = jnp.concatenate([scores,
                          jnp.full((NPAD - N,), _PAD_SCORE, jnp.float32)])

    mesh = plsc.VectorSubcoreMesh(core_axis_name="c", subcore_axis_name="s",
                                  num_cores=2, num_subcores=NSUB)
    run = pl.kernel(
        _nms_kernel,
        out_type=jax.ShapeDtypeStruct((MAX_DET * L,), jnp.float32),
        mesh=mesh,
        compiler_params=pltpu.CompilerParams(needs_layout_passes=False),
        scratch_types=[
            pltpu.VMEM((PER,), jnp.float32),
            pltpu.VMEM((PER,), jnp.float32),
            pltpu.VMEM((PER,), jnp.float32),
            pltpu.VMEM((PER,), jnp.float32),
            pltpu.VMEM((PER,), jnp.float32),
            pltpu.VMEM((PER,), jnp.float32),
            pltpu.HBM((2, 2, NSUB, L), jnp.float32),
            pltpu.VMEM((NSUB, L), jnp.float32),
            pltpu.VMEM((L,), jnp.float32),
            pltpu.VMEM((MAX_DET * L,), jnp.float32),
            pltpu.SemaphoreType.DMA,
        ],
    )
    r = run(x1, y1, x2, y2, sp).reshape(MAX_DET, L)
    return r[:, :4], r[:, 4], r[:, 5].astype(jnp.int32)
